# Initial kernel scaffold; baseline (speedup 1.0000x reference)
import jax
import jax.numpy as jnp
from jax.experimental import pallas as pl


def kernel(x):
    raise NotImplementedError("write your pallas kernel here")


# speedup vs baseline: 8.3281x; 8.3281x over previous
import jax
import jax.numpy as jnp
import numpy as np
from jax.experimental import pallas as pl
from jax.experimental.pallas import tpu as pltpu

_SCALE = 2.0 ** -1.5
_DBLK = 16


def _haar_ll_kernel(x_ref, p_ref, o_ref):
    p = p_ref[...]
    for k in range(_DBLK // 2):
        ev = pl.ds(0, 64, 2)
        od = pl.ds(1, 64, 2)
        s = (
            x_ref[0, 2 * k, ev, :]
            + x_ref[0, 2 * k, od, :]
            + x_ref[0, 2 * k + 1, ev, :]
            + x_ref[0, 2 * k + 1, od, :]
        )
        o_ref[0, k] = jnp.dot(s, p, preferred_element_type=jnp.float32)


@jax.jit
def kernel(x):
    B, C, D, H, W = x.shape
    n = B * C
    xr = x.reshape(n, D, H, W)
    rows = np.arange(W) // 2
    p = (rows[:, None] == np.arange(W // 2)[None, :]).astype(np.float32) * _SCALE
    p = jnp.asarray(p)

    grid = (n, D // _DBLK)
    out = pl.pallas_call(
        _haar_ll_kernel,
        grid=grid,
        in_specs=[
            pl.BlockSpec((1, _DBLK, H, W), lambda i, j: (i, j, 0, 0)),
            pl.BlockSpec((W, W // 2), lambda i, j: (0, 0)),
        ],
        out_specs=pl.BlockSpec(
            (1, _DBLK // 2, H // 2, W // 2), lambda i, j: (i, j, 0, 0)
        ),
        out_shape=jax.ShapeDtypeStruct((n, D // 2, H // 2, W // 2), x.dtype),
        compiler_params=pltpu.CompilerParams(
            dimension_semantics=("parallel", "arbitrary"),
        ),
    )(xr, p)
    return out.reshape(B, C, D // 2, H // 2, W // 2)


# speedup vs baseline: 12.2697x; 1.4733x over previous
import jax
import jax.numpy as jnp
import numpy as np
from jax.experimental import pallas as pl
from jax.experimental.pallas import tpu as pltpu

_SCALE = 2.0 ** -1.5
_DBLK = 32


def _haar_ll_kernel(x_ref, p_ref, o_ref):
    p = p_ref[...]
    for k in range(_DBLK // 2):
        ev = pl.ds(0, 64, 2)
        od = pl.ds(1, 64, 2)
        s = (
            x_ref[0, 2 * k, ev, :]
            + x_ref[0, 2 * k, od, :]
            + x_ref[0, 2 * k + 1, ev, :]
            + x_ref[0, 2 * k + 1, od, :]
        )
        o_ref[0, k] = jnp.dot(s, p, preferred_element_type=jnp.float32)


@jax.jit
def kernel(x):
    B, C, D, H, W = x.shape
    n = B * C
    xr = x.reshape(n, D, H, W)
    rows = np.arange(W) // 2
    p = (rows[:, None] == np.arange(W // 2)[None, :]).astype(np.float32) * _SCALE
    p = jnp.asarray(p)

    grid = (n, D // _DBLK)
    out = pl.pallas_call(
        _haar_ll_kernel,
        grid=grid,
        in_specs=[
            pl.BlockSpec((1, _DBLK, H, W), lambda i, j: (i, j, 0, 0)),
            pl.BlockSpec((W, W // 2), lambda i, j: (0, 0)),
        ],
        out_specs=pl.BlockSpec(
            (1, _DBLK // 2, H // 2, W // 2), lambda i, j: (i, j, 0, 0)
        ),
        out_shape=jax.ShapeDtypeStruct((n, D // 2, H // 2, W // 2), x.dtype),
        compiler_params=pltpu.CompilerParams(
            dimension_semantics=("parallel", "arbitrary"),
        ),
    )(xr, p)
    return out.reshape(B, C, D // 2, H // 2, W // 2)


# speedup vs baseline: 16.6937x; 1.3606x over previous
import jax
import jax.numpy as jnp
import numpy as np
from jax.experimental import pallas as pl
from jax.experimental.pallas import tpu as pltpu

_SCALE = 2.0 ** -1.5
_DBLK = 64


def _haar_ll_kernel(x_ref, p_ref, o_ref):
    p = p_ref[...]
    for k in range(_DBLK // 2):
        ev = pl.ds(0, 64, 2)
        od = pl.ds(1, 64, 2)
        s = (
            x_ref[0, 2 * k, ev, :]
            + x_ref[0, 2 * k, od, :]
            + x_ref[0, 2 * k + 1, ev, :]
            + x_ref[0, 2 * k + 1, od, :]
        )
        o_ref[0, k] = jnp.dot(s, p, preferred_element_type=jnp.float32)


@jax.jit
def kernel(x):
    B, C, D, H, W = x.shape
    n = B * C
    xr = x.reshape(n, D, H, W)
    rows = np.arange(W) // 2
    p = (rows[:, None] == np.arange(W // 2)[None, :]).astype(np.float32) * _SCALE
    p = jnp.asarray(p)

    grid = (n, D // _DBLK)
    out = pl.pallas_call(
        _haar_ll_kernel,
        grid=grid,
        in_specs=[
            pl.BlockSpec((1, _DBLK, H, W), lambda i, j: (i, j, 0, 0)),
            pl.BlockSpec((W, W // 2), lambda i, j: (0, 0)),
        ],
        out_specs=pl.BlockSpec(
            (1, _DBLK // 2, H // 2, W // 2), lambda i, j: (i, j, 0, 0)
        ),
        out_shape=jax.ShapeDtypeStruct((n, D // 2, H // 2, W // 2), x.dtype),
        compiler_params=pltpu.CompilerParams(
            dimension_semantics=("parallel", "arbitrary"),
        ),
    )(xr, p)
    return out.reshape(B, C, D // 2, H // 2, W // 2)


# speedup vs baseline: 17.5406x; 1.0507x over previous
import jax
import jax.numpy as jnp
import numpy as np
from jax.experimental import pallas as pl
from jax.experimental.pallas import tpu as pltpu

_SCALE = 2.0 ** -1.5
_DBLK = 128


def _haar_ll_kernel(x_ref, p_ref, o_ref):
    p = p_ref[...]
    for k in range(_DBLK // 2):
        ev = pl.ds(0, 64, 2)
        od = pl.ds(1, 64, 2)
        s = (
            x_ref[0, 2 * k, ev, :]
            + x_ref[0, 2 * k, od, :]
            + x_ref[0, 2 * k + 1, ev, :]
            + x_ref[0, 2 * k + 1, od, :]
        )
        o_ref[0, k] = jnp.dot(s, p, preferred_element_type=jnp.float32)


@jax.jit
def kernel(x):
    B, C, D, H, W = x.shape
    n = B * C
    xr = x.reshape(n, D, H, W)
    rows = np.arange(W) // 2
    p = (rows[:, None] == np.arange(W // 2)[None, :]).astype(np.float32) * _SCALE
    p = jnp.asarray(p)

    grid = (n, D // _DBLK)
    out = pl.pallas_call(
        _haar_ll_kernel,
        grid=grid,
        in_specs=[
            pl.BlockSpec((1, _DBLK, H, W), lambda i, j: (i, j, 0, 0)),
            pl.BlockSpec((W, W // 2), lambda i, j: (0, 0)),
        ],
        out_specs=pl.BlockSpec(
            (1, _DBLK // 2, H // 2, W // 2), lambda i, j: (i, j, 0, 0)
        ),
        out_shape=jax.ShapeDtypeStruct((n, D // 2, H // 2, W // 2), x.dtype),
        compiler_params=pltpu.CompilerParams(
            dimension_semantics=("parallel", "arbitrary"),
        ),
    )(xr, p)
    return out.reshape(B, C, D // 2, H // 2, W // 2)
